# baseline (speedup 1.0000x reference)
import functools

import jax
import jax.numpy as jnp
from jax import lax
from jax.experimental import pallas as pl
from jax.experimental.pallas import tpu as pltpu
from jax.experimental.pallas import tpu_sc as plsc

B, N, C = 8, 8192, 3
D = 128
S = 2048
K = 32
CG = D + C
COUT = CG + D


def _fps_body(x_ref, y_ref, z_ref, out_ref, dist_ref):
    x = x_ref[...]
    y = y_ref[...]
    z = z_ref[...]
    iota = jax.lax.broadcasted_iota(jnp.int32, (B, N), 1)
    dist_ref[...] = jnp.full((B, N), 1e10, dtype=jnp.float32)

    def step(t, far):
        out_ref[pl.ds(t, 1), :] = far.reshape(1, B)
        mask = iota == far
        cx = jnp.sum(jnp.where(mask, x, 0.0), axis=1, keepdims=True)
        cy = jnp.sum(jnp.where(mask, y, 0.0), axis=1, keepdims=True)
        cz = jnp.sum(jnp.where(mask, z, 0.0), axis=1, keepdims=True)
        dx = x - cx
        dy = y - cy
        dz = z - cz
        d = (dx * dx + dy * dy) + dz * dz
        dist = jnp.minimum(dist_ref[...], d)
        dist_ref[...] = dist
        m = jnp.max(dist, axis=1, keepdims=True)
        far = jnp.min(jnp.where(dist == m, iota, N), axis=1, keepdims=True)
        return far

    jax.lax.fori_loop(0, S, step, jnp.zeros((B, 1), jnp.int32))


def _fps(xyz, npoint):
    del npoint
    xyzT = jnp.swapaxes(xyz, 0, 2).swapaxes(1, 2)
    idx_sb = pl.pallas_call(
        _fps_body,
        grid=(1,),
        in_specs=[
            pl.BlockSpec((None, B, N), lambda i: (0, 0, 0)),
            pl.BlockSpec((None, B, N), lambda i: (1, 0, 0)),
            pl.BlockSpec((None, B, N), lambda i: (2, 0, 0)),
        ],
        out_specs=pl.BlockSpec((S, B), lambda i: (0, 0)),
        out_shape=jax.ShapeDtypeStruct((S, B), jnp.int32),
        scratch_shapes=[pltpu.VMEM((B, N), jnp.float32)],
    )(xyzT, xyzT, xyzT)
    return idx_sb.T


def _sqdist(src, dst):
    dist = -2.0 * jnp.matmul(src, jnp.swapaxes(dst, 1, 2))
    dist = dist + jnp.sum(src ** 2, axis=-1)[:, :, None]
    dist = dist + jnp.sum(dst ** 2, axis=-1)[:, None, :]
    return dist


def _gather(points, idx):
    return jax.vmap(lambda p, i: p[i])(points, idx)


RT = 256


def _topk_body(d_ref, idx_ref):
    iota = jax.lax.broadcasted_iota(jnp.int32, (RT, N), 1)
    inf = jnp.float32(jnp.inf)
    d = d_ref[...]
    v = jnp.full((RT, 1), -jnp.inf, jnp.float32)
    i = jnp.full((RT, 1), -1, jnp.int32)
    for j in range(K):
        gt = (d > v) | ((d == v) & (iota > i))
        v = jnp.min(jnp.where(gt, d, inf), axis=1, keepdims=True)
        i = jnp.min(jnp.where(gt & (d == v), iota, N), axis=1, keepdims=True)
        idx_ref[:, j:j + 1] = i


def _topk_idx(sqrdists):
    d2 = sqrdists.reshape(B * S, N)
    idx = pl.pallas_call(
        _topk_body,
        grid=(B * S // RT,),
        in_specs=[pl.BlockSpec((RT, N), lambda i: (i, 0))],
        out_specs=pl.BlockSpec((RT, K), lambda i: (i, 0)),
        out_shape=jax.ShapeDtypeStruct((B * S, K), jnp.int32),
    )(d2)
    return idx.reshape(B, S, K)


XPAD = 16
NW = 32
ROWS = B * S * K
RPW = ROWS // NW
CH = 128


def _sc_gather_kernel(pts_hbm, xyzp_hbm, idx_hbm, gp_hbm, gx_hbm,
                      idx_v, rows_v, rowsx_v, sem, semx):
    wid = lax.axis_index("c") * 16 + lax.axis_index("s")
    base0 = wid * RPW

    def chunk(i, _):
        base = base0 + i * CH
        pltpu.sync_copy(idx_hbm.at[pl.ds(base, CH)], idx_v)
        cp = pltpu.async_copy(pts_hbm.at[idx_v], rows_v, sem)
        cx = pltpu.async_copy(xyzp_hbm.at[idx_v], rowsx_v, semx)
        cp.wait()
        pltpu.sync_copy(rows_v, gp_hbm.at[pl.ds(base, CH)])
        cx.wait()
        pltpu.sync_copy(rowsx_v, gx_hbm.at[pl.ds(base, CH)])
        return 0

    lax.fori_loop(0, RPW // CH, chunk, 0)


def _sc_gather(points, xyz, idx):
    pts = points.reshape(B * N, D)
    xyzp = jnp.pad(xyz, ((0, 0), (0, 0), (0, D - C))).reshape(B * N, D)
    idxf = (idx + (jnp.arange(B, dtype=jnp.int32) * N)[:, None, None]).reshape(ROWS)
    mesh = plsc.VectorSubcoreMesh(core_axis_name="c", subcore_axis_name="s")
    f = functools.partial(
        pl.kernel,
        out_type=[
            jax.ShapeDtypeStruct((ROWS, D), jnp.float32),
            jax.ShapeDtypeStruct((ROWS, D), jnp.float32),
        ],
        mesh=mesh,
        scratch_types=[
            pltpu.VMEM((CH,), jnp.int32),
            pltpu.VMEM((CH, D), jnp.float32),
            pltpu.VMEM((CH, D), jnp.float32),
            pltpu.SemaphoreType.DMA,
            pltpu.SemaphoreType.DMA,
        ],
    )(_sc_gather_kernel)
    return f(pts, xyzp, idxf)


TS = 128


def _assemble_body(gp_ref, gx_ref, mgp_ref, mgx_ref, inv_ref, np_ref,
                   agp_ref, agx_ref, bgp_ref, bgx_ref, out_ref):
    inv = inv_ref[pl.program_id(0)]
    ngp = (gp_ref[...] - mgp_ref[...]) * inv * agp_ref[...] + bgp_ref[...]
    ngx = (gx_ref[:, :, :C] - mgx_ref[...]) * inv * agx_ref[...] + bgx_ref[...]
    npts = jnp.broadcast_to(np_ref[...][:, None, :], (TS, K, D))
    out_ref[...] = jnp.concatenate([ngp, ngx, npts], axis=-1)


def _assemble(gp4, gx4, mgp, mgx, inv_std, new_points, agp, agx, bgp, bgx):
    grid = (B, S // TS)
    return pl.pallas_call(
        _assemble_body,
        grid=grid,
        in_specs=[
            pl.BlockSpec((None, TS, K, D), lambda b, s: (b, s, 0, 0)),
            pl.BlockSpec((None, TS, K, D), lambda b, s: (b, s, 0, 0)),
            pl.BlockSpec((None, TS, 1, D), lambda b, s: (b, s, 0, 0)),
            pl.BlockSpec((None, TS, 1, C), lambda b, s: (b, s, 0, 0)),
            pl.BlockSpec(memory_space=pltpu.SMEM),
            pl.BlockSpec((None, TS, D), lambda b, s: (b, s, 0)),
            pl.BlockSpec((1, 1, D), lambda b, s: (0, 0, 0)),
            pl.BlockSpec((1, 1, C), lambda b, s: (0, 0, 0)),
            pl.BlockSpec((1, 1, D), lambda b, s: (0, 0, 0)),
            pl.BlockSpec((1, 1, C), lambda b, s: (0, 0, 0)),
        ],
        out_specs=pl.BlockSpec((None, TS, K, COUT), lambda b, s: (b, s, 0, 0)),
        out_shape=jax.ShapeDtypeStruct((B, S, K, COUT), jnp.float32),
    )(gp4, gx4, mgp, mgx, inv_std, new_points, agp, agx, bgp, bgx)


def kernel(xyz, points, affine_alpha, affine_beta):
    fps_idx = _fps(xyz, S)
    new_xyz = _gather(xyz, fps_idx)
    new_points = _gather(points, fps_idx)
    sqrdists = _sqdist(new_xyz, xyz)
    idx = _topk_idx(sqrdists)
    gp, gx = _sc_gather(points, xyz, idx)
    gp4 = gp.reshape(B, S, K, D)
    gxp4 = gx.reshape(B, S, K, D)
    gx4 = gxp4[..., :C]
    mgp = jnp.mean(gp4, axis=2, keepdims=True)
    mgx = jnp.mean(gx4, axis=2, keepdims=True)
    n = S * K * CG
    s2_gp = jnp.sum(gp4 * gp4, axis=(1, 2, 3)) - K * jnp.sum(mgp * mgp, axis=(1, 2, 3))
    s2_gx = jnp.sum(gx4 * gx4, axis=(1, 2, 3)) - K * jnp.sum(mgx * mgx, axis=(1, 2, 3))
    std = jnp.sqrt((s2_gp + s2_gx) / (n - 1))
    inv_std = 1.0 / (std + 1e-05)
    agp = affine_alpha.reshape(1, 1, CG)[:, :, :D]
    bgp = affine_beta.reshape(1, 1, CG)[:, :, :D]
    agx = affine_alpha.reshape(1, 1, CG)[:, :, D:]
    bgx = affine_beta.reshape(1, 1, CG)[:, :, D:]
    out = _assemble(gp4, gxp4, mgp, mgx, inv_std, new_points, agp, agx, bgp, bgx)
    return (new_xyz, out)


# speedup vs baseline: 1.3962x; 1.3962x over previous
import functools

import jax
import jax.numpy as jnp
from jax import lax
from jax.experimental import pallas as pl
from jax.experimental.pallas import tpu as pltpu
from jax.experimental.pallas import tpu_sc as plsc

B, N, C = 8, 8192, 3
D = 128
S = 2048
K = 32
CG = D + C
COUT = CG + D


def _fps_body(x_ref, y_ref, z_ref, out_ref, dist_ref):
    x = x_ref[...]
    y = y_ref[...]
    z = z_ref[...]
    iota = jax.lax.broadcasted_iota(jnp.int32, (B, N), 1)
    dist_ref[...] = jnp.full((B, N), 1e10, dtype=jnp.float32)

    def step(t, far):
        out_ref[pl.ds(t, 1), :] = far.reshape(1, B)
        mask = iota == far
        cx = jnp.sum(jnp.where(mask, x, 0.0), axis=1, keepdims=True)
        cy = jnp.sum(jnp.where(mask, y, 0.0), axis=1, keepdims=True)
        cz = jnp.sum(jnp.where(mask, z, 0.0), axis=1, keepdims=True)
        dx = x - cx
        dy = y - cy
        dz = z - cz
        d = (dx * dx + dy * dy) + dz * dz
        dist = jnp.minimum(dist_ref[...], d)
        dist_ref[...] = dist
        m = jnp.max(dist, axis=1, keepdims=True)
        far = jnp.min(jnp.where(dist == m, iota, N), axis=1, keepdims=True)
        return far

    jax.lax.fori_loop(0, S, step, jnp.zeros((B, 1), jnp.int32))


def _fps(xyz, npoint):
    del npoint
    xyzT = jnp.swapaxes(xyz, 0, 2).swapaxes(1, 2)
    idx_sb = pl.pallas_call(
        _fps_body,
        grid=(1,),
        in_specs=[
            pl.BlockSpec((None, B, N), lambda i: (0, 0, 0)),
            pl.BlockSpec((None, B, N), lambda i: (1, 0, 0)),
            pl.BlockSpec((None, B, N), lambda i: (2, 0, 0)),
        ],
        out_specs=pl.BlockSpec((S, B), lambda i: (0, 0)),
        out_shape=jax.ShapeDtypeStruct((S, B), jnp.int32),
        scratch_shapes=[pltpu.VMEM((B, N), jnp.float32)],
    )(xyzT, xyzT, xyzT)
    return idx_sb.T


def _sqdist(src, dst):
    dist = -2.0 * jnp.matmul(src, jnp.swapaxes(dst, 1, 2))
    dist = dist + jnp.sum(src ** 2, axis=-1)[:, :, None]
    dist = dist + jnp.sum(dst ** 2, axis=-1)[:, None, :]
    return dist


def _gather(points, idx):
    return jax.vmap(lambda p, i: p[i])(points, idx)


RT = 256


def _topk_body(d_ref, idx_ref, work_ref):
    iota = jax.lax.broadcasted_iota(jnp.int32, (RT, N), 1)
    inf = jnp.float32(jnp.inf)
    work_ref[...] = d_ref[...]
    loc = None
    for j in range(K):
        if loc is not None:
            work = jnp.where(iota == loc, inf, work_ref[...])
            work_ref[...] = work
        else:
            work = work_ref[...]
        m = jnp.min(work, axis=1, keepdims=True)
        loc = jnp.min(jnp.where(work == m, iota, N), axis=1, keepdims=True)
        idx_ref[:, j:j + 1] = loc


def _topk_idx(sqrdists):
    d2 = sqrdists.reshape(B * S, N)
    idx = pl.pallas_call(
        _topk_body,
        grid=(B * S // RT,),
        in_specs=[pl.BlockSpec((RT, N), lambda i: (i, 0))],
        out_specs=pl.BlockSpec((RT, K), lambda i: (i, 0)),
        out_shape=jax.ShapeDtypeStruct((B * S, K), jnp.int32),
        scratch_shapes=[pltpu.VMEM((RT, N), jnp.float32)],
    )(d2)
    return idx.reshape(B, S, K)


XPAD = 16
NW = 32
ROWS = B * S * K
RPW = ROWS // NW
CH = 128


def _sc_gather_kernel(pts_hbm, xyzp_hbm, idx_hbm, gp_hbm, gx_hbm,
                      idx_v, rows_v, rowsx_v, sem, semx):
    wid = lax.axis_index("c") * 16 + lax.axis_index("s")
    base0 = wid * RPW

    def chunk(i, _):
        base = base0 + i * CH
        pltpu.sync_copy(idx_hbm.at[pl.ds(base, CH)], idx_v)
        cp = pltpu.async_copy(pts_hbm.at[idx_v], rows_v, sem)
        cx = pltpu.async_copy(xyzp_hbm.at[idx_v], rowsx_v, semx)
        cp.wait()
        pltpu.sync_copy(rows_v, gp_hbm.at[pl.ds(base, CH)])
        cx.wait()
        pltpu.sync_copy(rowsx_v, gx_hbm.at[pl.ds(base, CH)])
        return 0

    lax.fori_loop(0, RPW // CH, chunk, 0)


def _sc_gather(points, xyz, idx):
    pts = points.reshape(B * N, D)
    xyzp = jnp.pad(xyz, ((0, 0), (0, 0), (0, D - C))).reshape(B * N, D)
    idxf = (idx + (jnp.arange(B, dtype=jnp.int32) * N)[:, None, None]).reshape(ROWS)
    mesh = plsc.VectorSubcoreMesh(core_axis_name="c", subcore_axis_name="s")
    f = functools.partial(
        pl.kernel,
        out_type=[
            jax.ShapeDtypeStruct((ROWS, D), jnp.float32),
            jax.ShapeDtypeStruct((ROWS, D), jnp.float32),
        ],
        mesh=mesh,
        scratch_types=[
            pltpu.VMEM((CH,), jnp.int32),
            pltpu.VMEM((CH, D), jnp.float32),
            pltpu.VMEM((CH, D), jnp.float32),
            pltpu.SemaphoreType.DMA,
            pltpu.SemaphoreType.DMA,
        ],
    )(_sc_gather_kernel)
    return f(pts, xyzp, idxf)


TS = 128


def _assemble_body(gp_ref, gx_ref, mgp_ref, mgx_ref, inv_ref, np_ref,
                   agp_ref, agx_ref, bgp_ref, bgx_ref, out_ref):
    inv = inv_ref[pl.program_id(0)]
    ngp = (gp_ref[...] - mgp_ref[...]) * inv * agp_ref[...] + bgp_ref[...]
    ngx = (gx_ref[:, :, :C] - mgx_ref[...]) * inv * agx_ref[...] + bgx_ref[...]
    npts = jnp.broadcast_to(np_ref[...][:, None, :], (TS, K, D))
    out_ref[...] = jnp.concatenate([ngp, ngx, npts], axis=-1)


def _assemble(gp4, gx4, mgp, mgx, inv_std, new_points, agp, agx, bgp, bgx):
    grid = (B, S // TS)
    return pl.pallas_call(
        _assemble_body,
        grid=grid,
        in_specs=[
            pl.BlockSpec((None, TS, K, D), lambda b, s: (b, s, 0, 0)),
            pl.BlockSpec((None, TS, K, D), lambda b, s: (b, s, 0, 0)),
            pl.BlockSpec((None, TS, 1, D), lambda b, s: (b, s, 0, 0)),
            pl.BlockSpec((None, TS, 1, C), lambda b, s: (b, s, 0, 0)),
            pl.BlockSpec(memory_space=pltpu.SMEM),
            pl.BlockSpec((None, TS, D), lambda b, s: (b, s, 0)),
            pl.BlockSpec((1, 1, D), lambda b, s: (0, 0, 0)),
            pl.BlockSpec((1, 1, C), lambda b, s: (0, 0, 0)),
            pl.BlockSpec((1, 1, D), lambda b, s: (0, 0, 0)),
            pl.BlockSpec((1, 1, C), lambda b, s: (0, 0, 0)),
        ],
        out_specs=pl.BlockSpec((None, TS, K, COUT), lambda b, s: (b, s, 0, 0)),
        out_shape=jax.ShapeDtypeStruct((B, S, K, COUT), jnp.float32),
    )(gp4, gx4, mgp, mgx, inv_std, new_points, agp, agx, bgp, bgx)


def kernel(xyz, points, affine_alpha, affine_beta):
    fps_idx = _fps(xyz, S)
    new_xyz = _gather(xyz, fps_idx)
    new_points = _gather(points, fps_idx)
    sqrdists = _sqdist(new_xyz, xyz)
    idx = _topk_idx(sqrdists)
    gp, gx = _sc_gather(points, xyz, idx)
    gp4 = gp.reshape(B, S, K, D)
    gxp4 = gx.reshape(B, S, K, D)
    gx4 = gxp4[..., :C]
    mgp = jnp.mean(gp4, axis=2, keepdims=True)
    mgx = jnp.mean(gx4, axis=2, keepdims=True)
    n = S * K * CG
    s2_gp = jnp.sum(gp4 * gp4, axis=(1, 2, 3)) - K * jnp.sum(mgp * mgp, axis=(1, 2, 3))
    s2_gx = jnp.sum(gx4 * gx4, axis=(1, 2, 3)) - K * jnp.sum(mgx * mgx, axis=(1, 2, 3))
    std = jnp.sqrt((s2_gp + s2_gx) / (n - 1))
    inv_std = 1.0 / (std + 1e-05)
    agp = affine_alpha.reshape(1, 1, CG)[:, :, :D]
    bgp = affine_beta.reshape(1, 1, CG)[:, :, :D]
    agx = affine_alpha.reshape(1, 1, CG)[:, :, D:]
    bgx = affine_beta.reshape(1, 1, CG)[:, :, D:]
    out = _assemble(gp4, gxp4, mgp, mgx, inv_std, new_points, agp, agx, bgp, bgx)
    return (new_xyz, out)


# speedup vs baseline: 1.4170x; 1.0149x over previous
import functools

import jax
import jax.numpy as jnp
from jax import lax
from jax.experimental import pallas as pl
from jax.experimental.pallas import tpu as pltpu
from jax.experimental.pallas import tpu_sc as plsc

B, N, C = 8, 8192, 3
D = 128
S = 2048
K = 32
CG = D + C
COUT = CG + D


def _fps_body(x_ref, y_ref, z_ref, out_ref, dist_ref):
    x = x_ref[...]
    y = y_ref[...]
    z = z_ref[...]
    iota = jax.lax.broadcasted_iota(jnp.int32, (B, N), 1)
    dist_ref[...] = jnp.full((B, N), 1e10, dtype=jnp.float32)

    def step(t, far):
        out_ref[pl.ds(t, 1), :] = far.reshape(1, B)
        mask = iota == far
        cx = jnp.sum(jnp.where(mask, x, 0.0), axis=1, keepdims=True)
        cy = jnp.sum(jnp.where(mask, y, 0.0), axis=1, keepdims=True)
        cz = jnp.sum(jnp.where(mask, z, 0.0), axis=1, keepdims=True)
        dx = x - cx
        dy = y - cy
        dz = z - cz
        d = (dx * dx + dy * dy) + dz * dz
        dist = jnp.minimum(dist_ref[...], d)
        dist_ref[...] = dist
        m = jnp.max(dist, axis=1, keepdims=True)
        far = jnp.min(jnp.where(dist == m, iota, N), axis=1, keepdims=True)
        return far

    jax.lax.fori_loop(0, S, step, jnp.zeros((B, 1), jnp.int32))


def _fps(xyz, npoint):
    del npoint
    xyzT = jnp.swapaxes(xyz, 0, 2).swapaxes(1, 2)
    idx_sb = pl.pallas_call(
        _fps_body,
        grid=(1,),
        in_specs=[
            pl.BlockSpec((None, B, N), lambda i: (0, 0, 0)),
            pl.BlockSpec((None, B, N), lambda i: (1, 0, 0)),
            pl.BlockSpec((None, B, N), lambda i: (2, 0, 0)),
        ],
        out_specs=pl.BlockSpec((S, B), lambda i: (0, 0)),
        out_shape=jax.ShapeDtypeStruct((S, B), jnp.int32),
        scratch_shapes=[pltpu.VMEM((B, N), jnp.float32)],
    )(xyzT, xyzT, xyzT)
    return idx_sb.T


def _sqdist(src, dst):
    dist = -2.0 * jnp.matmul(src, jnp.swapaxes(dst, 1, 2))
    dist = dist + jnp.sum(src ** 2, axis=-1)[:, :, None]
    dist = dist + jnp.sum(dst ** 2, axis=-1)[:, None, :]
    return dist


def _gather(points, idx):
    return jax.vmap(lambda p, i: p[i])(points, idx)


RT = 256


def _topk_body(q_ref, x_ref, q2_ref, x2_ref, idx_ref, work_ref):
    mm = jax.lax.dot_general(
        q_ref[...], x_ref[...], (((1,), (1,)), ((), ())),
        preferred_element_type=jnp.float32)
    work_ref[...] = (-2.0 * mm + q2_ref[...]) + x2_ref[...]
    iota = jax.lax.broadcasted_iota(jnp.int32, (RT, N), 1)
    inf = jnp.float32(jnp.inf)
    for j in range(K):
        work = work_ref[...]
        m = jnp.min(work, axis=1, keepdims=True)
        loc = jnp.min(jnp.where(work == m, iota, N), axis=1, keepdims=True)
        idx_ref[:, j:j + 1] = loc
        work_ref[...] = jnp.where(iota == loc, inf, work)


def _topk_idx(new_xyz, xyz):
    q2 = jnp.sum(new_xyz ** 2, axis=-1)[:, :, None]
    x2 = jnp.sum(xyz ** 2, axis=-1)[:, None, :]
    idx = pl.pallas_call(
        _topk_body,
        grid=(B, S // RT),
        in_specs=[
            pl.BlockSpec((None, RT, C), lambda b, s: (b, s, 0)),
            pl.BlockSpec((None, N, C), lambda b, s: (b, 0, 0)),
            pl.BlockSpec((None, RT, 1), lambda b, s: (b, s, 0)),
            pl.BlockSpec((None, 1, N), lambda b, s: (b, 0, 0)),
        ],
        out_specs=pl.BlockSpec((None, RT, K), lambda b, s: (b, s, 0)),
        out_shape=jax.ShapeDtypeStruct((B, S, K), jnp.int32),
        scratch_shapes=[pltpu.VMEM((RT, N), jnp.float32)],
    )(new_xyz, xyz, q2, x2)
    return idx


XPAD = 16
NW = 32
ROWS = B * S * K
RPW = ROWS // NW
CH = 128


def _sc_gather_kernel(pts_hbm, xyzp_hbm, idx_hbm, gp_hbm, gx_hbm,
                      idx_v, rows_v, rowsx_v, sem, semx):
    wid = lax.axis_index("c") * 16 + lax.axis_index("s")
    base0 = wid * RPW

    def chunk(i, _):
        base = base0 + i * CH
        pltpu.sync_copy(idx_hbm.at[pl.ds(base, CH)], idx_v)
        cp = pltpu.async_copy(pts_hbm.at[idx_v], rows_v, sem)
        cx = pltpu.async_copy(xyzp_hbm.at[idx_v], rowsx_v, semx)
        cp.wait()
        pltpu.sync_copy(rows_v, gp_hbm.at[pl.ds(base, CH)])
        cx.wait()
        pltpu.sync_copy(rowsx_v, gx_hbm.at[pl.ds(base, CH)])
        return 0

    lax.fori_loop(0, RPW // CH, chunk, 0)


def _sc_gather(points, xyz, idx):
    pts = points.reshape(B * N, D)
    xyzp = jnp.pad(xyz, ((0, 0), (0, 0), (0, D - C))).reshape(B * N, D)
    idxf = (idx + (jnp.arange(B, dtype=jnp.int32) * N)[:, None, None]).reshape(ROWS)
    mesh = plsc.VectorSubcoreMesh(core_axis_name="c", subcore_axis_name="s")
    f = functools.partial(
        pl.kernel,
        out_type=[
            jax.ShapeDtypeStruct((ROWS, D), jnp.float32),
            jax.ShapeDtypeStruct((ROWS, D), jnp.float32),
        ],
        mesh=mesh,
        scratch_types=[
            pltpu.VMEM((CH,), jnp.int32),
            pltpu.VMEM((CH, D), jnp.float32),
            pltpu.VMEM((CH, D), jnp.float32),
            pltpu.SemaphoreType.DMA,
            pltpu.SemaphoreType.DMA,
        ],
    )(_sc_gather_kernel)
    return f(pts, xyzp, idxf)


TS = 128


def _assemble_body(gp_ref, gx_ref, mgp_ref, mgx_ref, inv_ref, np_ref,
                   agp_ref, agx_ref, bgp_ref, bgx_ref, out_ref):
    inv = inv_ref[pl.program_id(0)]
    ngp = (gp_ref[...] - mgp_ref[...]) * inv * agp_ref[...] + bgp_ref[...]
    ngx = (gx_ref[:, :, :C] - mgx_ref[...]) * inv * agx_ref[...] + bgx_ref[...]
    npts = jnp.broadcast_to(np_ref[...][:, None, :], (TS, K, D))
    out_ref[...] = jnp.concatenate([ngp, ngx, npts], axis=-1)


def _assemble(gp4, gx4, mgp, mgx, inv_std, new_points, agp, agx, bgp, bgx):
    grid = (B, S // TS)
    return pl.pallas_call(
        _assemble_body,
        grid=grid,
        in_specs=[
            pl.BlockSpec((None, TS, K, D), lambda b, s: (b, s, 0, 0)),
            pl.BlockSpec((None, TS, K, D), lambda b, s: (b, s, 0, 0)),
            pl.BlockSpec((None, TS, 1, D), lambda b, s: (b, s, 0, 0)),
            pl.BlockSpec((None, TS, 1, C), lambda b, s: (b, s, 0, 0)),
            pl.BlockSpec(memory_space=pltpu.SMEM),
            pl.BlockSpec((None, TS, D), lambda b, s: (b, s, 0)),
            pl.BlockSpec((1, 1, D), lambda b, s: (0, 0, 0)),
            pl.BlockSpec((1, 1, C), lambda b, s: (0, 0, 0)),
            pl.BlockSpec((1, 1, D), lambda b, s: (0, 0, 0)),
            pl.BlockSpec((1, 1, C), lambda b, s: (0, 0, 0)),
        ],
        out_specs=pl.BlockSpec((None, TS, K, COUT), lambda b, s: (b, s, 0, 0)),
        out_shape=jax.ShapeDtypeStruct((B, S, K, COUT), jnp.float32),
    )(gp4, gx4, mgp, mgx, inv_std, new_points, agp, agx, bgp, bgx)


def kernel(xyz, points, affine_alpha, affine_beta):
    fps_idx = _fps(xyz, S)
    new_xyz = _gather(xyz, fps_idx)
    new_points = _gather(points, fps_idx)
    idx = _topk_idx(new_xyz, xyz)
    gp, gx = _sc_gather(points, xyz, idx)
    gp4 = gp.reshape(B, S, K, D)
    gxp4 = gx.reshape(B, S, K, D)
    gx4 = gxp4[..., :C]
    mgp = jnp.mean(gp4, axis=2, keepdims=True)
    mgx = jnp.mean(gx4, axis=2, keepdims=True)
    n = S * K * CG
    s2_gp = jnp.sum(gp4 * gp4, axis=(1, 2, 3)) - K * jnp.sum(mgp * mgp, axis=(1, 2, 3))
    s2_gx = jnp.sum(gx4 * gx4, axis=(1, 2, 3)) - K * jnp.sum(mgx * mgx, axis=(1, 2, 3))
    std = jnp.sqrt((s2_gp + s2_gx) / (n - 1))
    inv_std = 1.0 / (std + 1e-05)
    agp = affine_alpha.reshape(1, 1, CG)[:, :, :D]
    bgp = affine_beta.reshape(1, 1, CG)[:, :, :D]
    agx = affine_alpha.reshape(1, 1, CG)[:, :, D:]
    bgx = affine_beta.reshape(1, 1, CG)[:, :, D:]
    out = _assemble(gp4, gxp4, mgp, mgx, inv_std, new_points, agp, agx, bgp, bgx)
    return (new_xyz, out)
